# baseline (speedup 1.0000x reference)
import functools

import jax
import jax.numpy as jnp
from jax import lax
from jax.experimental import pallas as pl
from jax.experimental.pallas import tpu as pltpu
from jax.experimental.pallas import tpu_sc as plsc

N_NODES = 10000
N_EDGES = 320000
D_IN = 128
D_HID = 256

NC = 2
NS = 16
N_WORKERS = NC * NS

CHUNK = 128
N_CHUNKS = 80
E_PAD = N_WORKERS * N_CHUNKS * CHUNK
N_PAD = 10240
ROWS_PER_TILE = N_PAD // NS
NBUF = 2
NI = 4


def _sc_agg(x, src_blk, dst_blk):
    mesh = plsc.VectorSubcoreMesh(core_axis_name="c", subcore_axis_name="s")

    @functools.partial(
        pl.kernel,
        out_type=jax.ShapeDtypeStruct((NC, N_PAD, D_IN), jnp.float32),
        mesh=mesh,
        scratch_types=[
            pltpu.VMEM((NI, CHUNK), jnp.int32),
            pltpu.VMEM((NI, CHUNK), jnp.int32),
            [pltpu.VMEM((CHUNK, D_IN), jnp.float32) for _ in range(NBUF)],
            pltpu.VMEM_SHARED((N_PAD, D_IN), jnp.float32),
            [pltpu.SemaphoreType.DMA for _ in range(NI)],
            [pltpu.SemaphoreType.DMA for _ in range(NI)],
            [pltpu.SemaphoreType.DMA for _ in range(NBUF)],
        ],
    )
    def k(x_hbm, src_hbm, dst_hbm, out_hbm, sbuf, dbuf, rows, agg_sh,
          isems, dsems, gsems):
        cid = lax.axis_index("c")
        sid = lax.axis_index("s")
        wid = sid * NC + cid

        def idx_start(j, u):
            pltpu.async_copy(src_hbm.at[wid, j], sbuf.at[u], isems[u])
            pltpu.async_copy(dst_hbm.at[wid, j], dbuf.at[u], dsems[u])

        def gather_start(u, b):
            pltpu.async_copy(x_hbm.at[sbuf.at[u]], rows[b], gsems[b])

        def gather_wait(u, b):
            pltpu.make_async_copy(x_hbm.at[sbuf.at[u]], rows[b],
                                  gsems[b]).wait()

        def idx_wait(j, u, which):
            sems, buf, hbm = ((isems, sbuf, src_hbm) if which == 0 else
                              (dsems, dbuf, dst_hbm))
            pltpu.make_async_copy(hbm.at[wid, j], buf.at[u], sems[u]).wait()

        for u in range(NI):
            idx_start(u, u)

        def zrow(r, carry):
            for c in range(D_IN // 16):
                rows[0][r, pl.ds(c * 16, 16)] = jnp.zeros((16,), jnp.float32)
            return carry
        lax.fori_loop(0, CHUNK, zrow, 0)
        nbase = sid * ROWS_PER_TILE
        for j in range(ROWS_PER_TILE // CHUNK):
            pltpu.sync_copy(rows[0], agg_sh.at[pl.ds(nbase + j * CHUNK, CHUNK)])

        idx_wait(0, 0, 0)
        gather_start(0, 0)

        plsc.subcore_barrier()

        def body(t, carry):
            g = NI * t
            for u in range(NI):
                j = g + u
                idx_wait(j + 1, (u + 1) % NI, 0)
                gather_start((u + 1) % NI, (u + 1) % NBUF)
                gather_wait(u % NI, u % NBUF)
                idx_wait(j, u, 1)
                pltpu.sync_copy(rows[u % NBUF], agg_sh.at[dbuf.at[u]],
                                add=True)
                idx_start(j + NI, u)
            return carry
        lax.fori_loop(0, N_CHUNKS // NI - 1, body, 0)

        for u in range(NI):
            j = N_CHUNKS - NI + u
            if j + 1 < N_CHUNKS:
                idx_wait(j + 1, (u + 1) % NI, 0)
                gather_start((u + 1) % NI, (u + 1) % NBUF)
            gather_wait(u % NI, u % NBUF)
            idx_wait(j, u, 1)
            pltpu.sync_copy(rows[u % NBUF], agg_sh.at[dbuf.at[u]], add=True)

        plsc.subcore_barrier()
        pltpu.sync_copy(agg_sh.at[pl.ds(nbase, ROWS_PER_TILE)],
                        out_hbm.at[cid, pl.ds(nbase, ROWS_PER_TILE)])

    return k(x, src_blk, dst_blk)


BLK = 1000


def _mlp_body(x_ref, a0_ref, a1_ref, w1_ref, b1_ref, w2_ref, b2_ref, o_ref):
    feat = x_ref[...] + a0_ref[...] + a1_ref[...]
    h = lax.dot_general(feat, w1_ref[...], (((1,), (1,)), ((), ())),
                        preferred_element_type=jnp.float32)
    h = jnp.maximum(h + b1_ref[...], 0.0)
    o = lax.dot_general(h, w2_ref[...], (((1,), (1,)), ((), ())),
                        preferred_element_type=jnp.float32)
    o_ref[...] = o + b2_ref[...]


def _mlp(x, a0, a1, W1, b1, W2, b2):
    return pl.pallas_call(
        _mlp_body,
        grid=(N_NODES // BLK,),
        in_specs=[
            pl.BlockSpec((BLK, D_IN), lambda i: (i, 0)),
            pl.BlockSpec((BLK, D_IN), lambda i: (i, 0)),
            pl.BlockSpec((BLK, D_IN), lambda i: (i, 0)),
            pl.BlockSpec((D_HID, D_IN), lambda i: (0, 0)),
            pl.BlockSpec((1, D_HID), lambda i: (0, 0)),
            pl.BlockSpec((D_IN, D_HID), lambda i: (0, 0)),
            pl.BlockSpec((1, D_IN), lambda i: (0, 0)),
        ],
        out_specs=pl.BlockSpec((BLK, D_IN), lambda i: (i, 0)),
        out_shape=jax.ShapeDtypeStruct((N_NODES, D_IN), jnp.float32),
    )(x, a0, a1, W1, b1.reshape(1, D_HID), W2, b2.reshape(1, D_IN))


def kernel(x, edge_index, W1, b1, W2, b2):
    src = edge_index[0].astype(jnp.int32)
    dst = edge_index[1].astype(jnp.int32)
    n_extra = E_PAD - N_EDGES
    src_blk = jnp.concatenate(
        [src, jnp.zeros((n_extra,), jnp.int32)]).reshape(
            N_WORKERS, N_CHUNKS, CHUNK)
    dst_blk = jnp.concatenate(
        [dst, jnp.full((n_extra,), N_PAD - 1, jnp.int32)]).reshape(
            N_WORKERS, N_CHUNKS, CHUNK)
    agg = _sc_agg(x, src_blk, dst_blk)
    return _mlp(x, agg[0], agg[1], W1, b1.reshape(-1), W2, b2)
